# Initial kernel scaffold; baseline (speedup 1.0000x reference)
import jax
import jax.numpy as jnp
from jax.experimental import pallas as pl


def kernel(x, edge_index, W_enc, W_l0, W_l1, W_fc):
    raise NotImplementedError("write your pallas kernel here")


# speedup vs baseline: 7.6137x; 7.6137x over previous
import functools

import jax
import jax.numpy as jnp
from jax import lax
from jax.experimental import pallas as pl
from jax.experimental.pallas import tpu as pltpu
from jax.experimental.pallas import tpu_sc as plsc

N = 10000
E = 320000
IN_CH = 128
D = 33
OUT_CH = 16
T = 4
STEP = 0.1
VTH = 1.0

DP = 128
NP = 10240
EP = 327680
IDX_ROWS = EP // 128
NW = 32
ROWS_PER_W = IDX_ROWS // NW
G = 2
DA = 48
GROUPS = ROWS_PER_W // G
ZROWS = NP // 16
BR = 512

def _sc_agg_body(table, src2, dst2, out, srcb, dstb, rowsb, acc, sem):
    c = lax.axis_index("c")
    s = lax.axis_index("s")

    def _z(i, _):
        zero16 = jnp.zeros((16,), jnp.float32)
        for k in range(DP // 16):
            rowsb[i, pl.ds(k * 16, 16)] = zero16
        return _
    lax.fori_loop(0, 128, _z, None)
    for k in range(ZROWS // 128):
        pltpu.sync_copy(rowsb.at[pl.ds(0, 128)],
                        acc.at[pl.ds(s * ZROWS + k * 128, 128)])
    plsc.subcore_barrier()

    def _group(g, _):
        base = (c * 16 + s) * ROWS_PER_W + g * G
        pltpu.sync_copy(src2.at[pl.ds(base, G)], srcb)
        pltpu.sync_copy(dst2.at[pl.ds(base, G)], dstb)
        descs = [
            pltpu.async_copy(
                table.at[srcb.at[j]], rowsb.at[pl.ds(j * 128, 128)], sem)
            for j in range(G)
        ]
        for d in descs:
            d.wait()
        for j in range(G):
            pltpu.sync_copy(
                rowsb.at[pl.ds(j * 128, 128)],
                acc.at[dstb.at[j]], add=True)
        return _
    lax.fori_loop(0, GROUPS, _group, None)

    plsc.subcore_barrier()
    pltpu.sync_copy(acc.at[pl.ds(s * ZROWS, ZROWS)],
                    out.at[c].at[pl.ds(s * ZROWS, ZROWS)])


@functools.lru_cache(maxsize=1)
def _sc_agg_fn():
    mesh = plsc.VectorSubcoreMesh(
        core_axis_name="c", subcore_axis_name="s", num_cores=2,
        num_subcores=16)
    return pl.kernel(
        _sc_agg_body,
        out_type=jax.ShapeDtypeStruct((2, NP, DP), jnp.float32),
        mesh=mesh,
        scratch_types=[
            pltpu.VMEM((G, 128), jnp.int32),
            pltpu.VMEM((G, 128), jnp.int32),
            pltpu.VMEM((G * 128, DP), jnp.float32),
            pltpu.VMEM_SHARED((NP, DP), jnp.float32),
            pltpu.SemaphoreType.DMA,
        ],
    )


def _sc_agg(table, src2, dst2):
    return _sc_agg_fn()(table, src2, dst2)


def _enc_body(x_ref, w_ref, o_ref):
    h = jnp.dot(x_ref[...], w_ref[...], preferred_element_type=jnp.float32)
    col = lax.broadcasted_iota(jnp.int32, (BR, DP), 1)
    o_ref[...] = jnp.where(col == D, 1.0, h)


def _tc_encode(x_pad, W_encp):
    return pl.pallas_call(
        _enc_body,
        grid=(NP // BR,),
        in_specs=[
            pl.BlockSpec((BR, IN_CH), lambda i: (i, 0)),
            pl.BlockSpec((IN_CH, DP), lambda i: (0, 0)),
        ],
        out_specs=pl.BlockSpec((BR, DP), lambda i: (i, 0)),
        out_shape=jax.ShapeDtypeStruct((NP, DP), jnp.float32),
    )(x_pad, W_encp)


def _step_body(raw_ref, u_ref, w_ref, hh_ref, un_ref):
    r = raw_ref[0] + raw_ref[1]
    col = lax.broadcasted_iota(jnp.int32, (BR, DP), 1)
    deg = jnp.maximum(r[:, D:D + 1], 1.0)
    agg = jnp.where(col < D, r / deg, 0.0)
    v = agg
    s = (v >= VTH).astype(jnp.float32)
    ssum = s
    for _ in range(T - 1):
        v = v - s + agg
        s = (v >= VTH).astype(jnp.float32)
        ssum = ssum + s
    rate = ssum * (1.0 / T)
    t = u_ref[...] + STEP * rate
    t = jnp.where((col >= 1) & (col < D), t, 0.0)
    n = jnp.sqrt(jnp.maximum(jnp.sum(t * t, axis=1, keepdims=True), 1e-12))
    en = jnp.exp(n)
    ien = 1.0 / en
    ch = 0.5 * (en + ien)
    sh = 0.5 * (en - ien)
    zs = sh / n * t
    x0 = jnp.maximum(ch, 1.0 + 1e-7)
    nn = jnp.sqrt(jnp.maximum(jnp.sum(zs * zs, axis=1, keepdims=True), 1e-12))
    d = jnp.log(x0 + jnp.sqrt((x0 - 1.0) * (x0 + 1.0)))
    un = d * zs / nn
    hh = jnp.dot(un, w_ref[...], preferred_element_type=jnp.float32)
    hh_ref[...] = jnp.where(col == D, 1.0, hh)
    un_ref[...] = un


def _tc_step(raw, u_prev, Wp):
    return pl.pallas_call(
        _step_body,
        grid=(NP // BR,),
        in_specs=[
            pl.BlockSpec((2, BR, DP), lambda i: (0, i, 0)),
            pl.BlockSpec((BR, DP), lambda i: (i, 0)),
            pl.BlockSpec((DP, DP), lambda i: (0, 0)),
        ],
        out_specs=[
            pl.BlockSpec((BR, DP), lambda i: (i, 0)),
            pl.BlockSpec((BR, DP), lambda i: (i, 0)),
        ],
        out_shape=[
            jax.ShapeDtypeStruct((NP, DP), jnp.float32),
            jax.ShapeDtypeStruct((NP, DP), jnp.float32),
        ],
    )(raw, u_prev, Wp)


def kernel(x, edge_index, W_enc, W_l0, W_l1, W_fc):
    src = edge_index[0]
    dst = edge_index[1]
    npad = EP - E
    pad_src = jnp.arange(npad, dtype=jnp.int32) % N
    pad_dst = N + jnp.arange(npad, dtype=jnp.int32) % (NP - N)
    src2 = jnp.concatenate([src, pad_src]).reshape(IDX_ROWS, 128)
    dst2 = jnp.concatenate([dst, pad_dst]).reshape(IDX_ROWS, 128)

    x_pad = jnp.pad(x, ((0, NP - N), (0, 0)))
    W_encp = jnp.pad(W_enc, ((0, 0), (0, DP - D)))
    W0p = jnp.pad(W_l0, ((0, DP - D), (0, DP - D)))
    W1p = jnp.pad(W_l1, ((0, DP - D), (0, DP - D)))
    Wfcp = jnp.pad(W_fc, ((0, DP - D), (0, DP - OUT_CH)))

    h = _tc_encode(x_pad, W_encp)
    raw = _sc_agg(h, src2, dst2)
    u = jnp.zeros((NP, DP), jnp.float32)
    hh, u = _tc_step(raw, u, W0p)
    raw = _sc_agg(hh, src2, dst2)
    hh, u = _tc_step(raw, u, W1p)
    raw = _sc_agg(hh, src2, dst2)
    outp, _ = _tc_step(raw, u, Wfcp)
    return outp[:N, :OUT_CH]


# speedup vs baseline: 9.7829x; 1.2849x over previous
import functools

import jax
import jax.numpy as jnp
from jax import lax
from jax.experimental import pallas as pl
from jax.experimental.pallas import tpu as pltpu
from jax.experimental.pallas import tpu_sc as plsc

N = 10000
E = 320000
IN_CH = 128
D = 33
OUT_CH = 16
T = 4
STEP = 0.1
VTH = 1.0

DP = 128
NP = 10240
EP = 327680
IDX_ROWS = EP // 128
NW = 32
ROWS_PER_W = IDX_ROWS // NW
CHUNK = 16
NCH = ROWS_PER_W // CHUNK
ZROWS = NP // 16
BR = 512

def _sc_agg_body(table, src2, dst2, out, srcb, dstb, rowsa, rowsb, acc,
                 gsema, gsemb, ssema, ssemb):
    c = lax.axis_index("c")
    s = lax.axis_index("s")

    def _z(i, _):
        zero16 = jnp.zeros((16,), jnp.float32)
        for k in range(DP // 16):
            rowsa[i, pl.ds(k * 16, 16)] = zero16
        return _
    lax.fori_loop(0, 128, _z, None)
    for k in range(ZROWS // 128):
        pltpu.sync_copy(rowsa, acc.at[pl.ds(s * ZROWS + k * 128, 128)])
    plsc.subcore_barrier()

    tbase = (c * 16 + s) * ROWS_PER_W

    def _chunk(ch, _):
        cb = tbase + ch * CHUNK
        pltpu.sync_copy(src2.at[pl.ds(cb, CHUNK)], srcb)
        pltpu.sync_copy(dst2.at[pl.ds(cb, CHUNK)], dstb)
        dg = {}
        dsc = {}
        dg[0] = pltpu.async_copy(table.at[srcb.at[0]], rowsa, gsema)
        for j in range(CHUNK):
            even = (j % 2 == 0)
            rows_j = rowsa if even else rowsb
            ssem_j = ssema if even else ssemb
            dg[j].wait()
            dsc[j] = pltpu.async_copy(rows_j, acc.at[dstb.at[j]], ssem_j,
                                      add=True)
            if j < CHUNK - 1:
                rows_n = rowsb if even else rowsa
                gsem_n = gsemb if even else gsema
                if j >= 1:
                    dsc[j - 1].wait()
                dg[j + 1] = pltpu.async_copy(
                    table.at[srcb.at[j + 1]], rows_n, gsem_n)
        dsc[CHUNK - 2].wait()
        dsc[CHUNK - 1].wait()
        return _
    lax.fori_loop(0, NCH, _chunk, None)

    plsc.subcore_barrier()
    pltpu.sync_copy(acc.at[pl.ds(s * ZROWS, ZROWS)],
                    out.at[c].at[pl.ds(s * ZROWS, ZROWS)])


@functools.lru_cache(maxsize=1)
def _sc_agg_fn():
    mesh = plsc.VectorSubcoreMesh(
        core_axis_name="c", subcore_axis_name="s", num_cores=2,
        num_subcores=16)
    return pl.kernel(
        _sc_agg_body,
        out_type=jax.ShapeDtypeStruct((2, NP, DP), jnp.float32),
        mesh=mesh,
        scratch_types=[
            pltpu.VMEM((CHUNK, 128), jnp.int32),
            pltpu.VMEM((CHUNK, 128), jnp.int32),
            pltpu.VMEM((128, DP), jnp.float32),
            pltpu.VMEM((128, DP), jnp.float32),
            pltpu.VMEM_SHARED((NP, DP), jnp.float32),
            pltpu.SemaphoreType.DMA,
            pltpu.SemaphoreType.DMA,
            pltpu.SemaphoreType.DMA,
            pltpu.SemaphoreType.DMA,
        ],
    )


def _sc_agg(table, src2, dst2):
    return _sc_agg_fn()(table, src2, dst2)


def _enc_body(x_ref, w_ref, o_ref):
    h = jnp.dot(x_ref[...], w_ref[...], preferred_element_type=jnp.float32)
    col = lax.broadcasted_iota(jnp.int32, (BR, DP), 1)
    o_ref[...] = jnp.where(col == D, 1.0, h)


def _tc_encode(x_pad, W_encp):
    return pl.pallas_call(
        _enc_body,
        grid=(NP // BR,),
        in_specs=[
            pl.BlockSpec((BR, IN_CH), lambda i: (i, 0)),
            pl.BlockSpec((IN_CH, DP), lambda i: (0, 0)),
        ],
        out_specs=pl.BlockSpec((BR, DP), lambda i: (i, 0)),
        out_shape=jax.ShapeDtypeStruct((NP, DP), jnp.float32),
    )(x_pad, W_encp)


def _step_body(raw_ref, u_ref, w_ref, hh_ref, un_ref):
    r = raw_ref[0] + raw_ref[1]
    col = lax.broadcasted_iota(jnp.int32, (BR, DP), 1)
    deg = jnp.maximum(r[:, D:D + 1], 1.0)
    agg = jnp.where(col < D, r / deg, 0.0)
    v = agg
    s = (v >= VTH).astype(jnp.float32)
    ssum = s
    for _ in range(T - 1):
        v = v - s + agg
        s = (v >= VTH).astype(jnp.float32)
        ssum = ssum + s
    rate = ssum * (1.0 / T)
    t = u_ref[...] + STEP * rate
    t = jnp.where((col >= 1) & (col < D), t, 0.0)
    n = jnp.sqrt(jnp.maximum(jnp.sum(t * t, axis=1, keepdims=True), 1e-12))
    en = jnp.exp(n)
    ien = 1.0 / en
    ch = 0.5 * (en + ien)
    sh = 0.5 * (en - ien)
    zs = sh / n * t
    x0 = jnp.maximum(ch, 1.0 + 1e-7)
    nn = jnp.sqrt(jnp.maximum(jnp.sum(zs * zs, axis=1, keepdims=True), 1e-12))
    d = jnp.log(x0 + jnp.sqrt((x0 - 1.0) * (x0 + 1.0)))
    un = d * zs / nn
    hh = jnp.dot(un, w_ref[...], preferred_element_type=jnp.float32)
    hh_ref[...] = jnp.where(col == D, 1.0, hh)
    un_ref[...] = un


def _tc_step(raw, u_prev, Wp):
    return pl.pallas_call(
        _step_body,
        grid=(NP // BR,),
        in_specs=[
            pl.BlockSpec((2, BR, DP), lambda i: (0, i, 0)),
            pl.BlockSpec((BR, DP), lambda i: (i, 0)),
            pl.BlockSpec((DP, DP), lambda i: (0, 0)),
        ],
        out_specs=[
            pl.BlockSpec((BR, DP), lambda i: (i, 0)),
            pl.BlockSpec((BR, DP), lambda i: (i, 0)),
        ],
        out_shape=[
            jax.ShapeDtypeStruct((NP, DP), jnp.float32),
            jax.ShapeDtypeStruct((NP, DP), jnp.float32),
        ],
    )(raw, u_prev, Wp)


def kernel(x, edge_index, W_enc, W_l0, W_l1, W_fc):
    src = edge_index[0]
    dst = edge_index[1]
    npad = EP - E
    pad_src = jnp.arange(npad, dtype=jnp.int32) % N
    pad_dst = N + jnp.arange(npad, dtype=jnp.int32) % (NP - N)
    src2 = jnp.concatenate([src, pad_src]).reshape(IDX_ROWS, 128)
    dst2 = jnp.concatenate([dst, pad_dst]).reshape(IDX_ROWS, 128)

    x_pad = jnp.pad(x, ((0, NP - N), (0, 0)))
    W_encp = jnp.pad(W_enc, ((0, 0), (0, DP - D)))
    W0p = jnp.pad(W_l0, ((0, DP - D), (0, DP - D)))
    W1p = jnp.pad(W_l1, ((0, DP - D), (0, DP - D)))
    Wfcp = jnp.pad(W_fc, ((0, DP - D), (0, DP - OUT_CH)))

    h = _tc_encode(x_pad, W_encp)
    raw = _sc_agg(h, src2, dst2)
    u = jnp.zeros((NP, DP), jnp.float32)
    hh, u = _tc_step(raw, u, W0p)
    raw = _sc_agg(hh, src2, dst2)
    hh, u = _tc_step(raw, u, W1p)
    raw = _sc_agg(hh, src2, dst2)
    outp, _ = _tc_step(raw, u, Wfcp)
    return outp[:N, :OUT_CH]


# speedup vs baseline: 11.7893x; 1.2051x over previous
import functools

import jax
import jax.numpy as jnp
from jax import lax
from jax.experimental import pallas as pl
from jax.experimental import layout as jlayout
from jax.experimental.pallas import tpu as pltpu
from jax.experimental.pallas import tpu_sc as plsc

N = 10000
E = 320000
IN_CH = 128
D = 33
OUT_CH = 16
T = 4
STEP = 0.1
VTH = 1.0

DW = 48
NP = 10240
EP = 327680
IDX_ROWS = EP // 128
NW = 32
ROWS_PER_W = IDX_ROWS // NW
CHUNK = 16
NCH = ROWS_PER_W // CHUNK
ZROWS = NP // 16
BR = 512

_LINEAR16 = jlayout.Layout(major_to_minor=(0, 1), tiling=((16,),))


def _sc_agg_body(table, src2, dst2, out, srcb, dstb, rowsa, rowsb, acc,
                 gsema, gsemb, ssema, ssemb):
    c = lax.axis_index("c")
    s = lax.axis_index("s")

    def _z(i, _):
        zero16 = jnp.zeros((16,), jnp.float32)
        for k in range(DW // 16):
            rowsa[i, pl.ds(k * 16, 16)] = zero16
        return _
    lax.fori_loop(0, 128, _z, None)
    for k in range(ZROWS // 128):
        pltpu.sync_copy(rowsa, acc.at[pl.ds(s * ZROWS + k * 128, 128)])
    plsc.subcore_barrier()

    tbase = (c * 16 + s) * ROWS_PER_W

    def _chunk(ch, _):
        cb = tbase + ch * CHUNK
        pltpu.sync_copy(src2.at[pl.ds(cb, CHUNK)], srcb)
        pltpu.sync_copy(dst2.at[pl.ds(cb, CHUNK)], dstb)
        dg = {}
        dsc = {}
        dg[0] = pltpu.async_copy(table.at[srcb.at[0]], rowsa, gsema)
        for j in range(CHUNK):
            even = (j % 2 == 0)
            rows_j = rowsa if even else rowsb
            ssem_j = ssema if even else ssemb
            dg[j].wait()
            dsc[j] = pltpu.async_copy(rows_j, acc.at[dstb.at[j]], ssem_j,
                                      add=True)
            if j < CHUNK - 1:
                rows_n = rowsb if even else rowsa
                gsem_n = gsemb if even else gsema
                if j >= 1:
                    dsc[j - 1].wait()
                dg[j + 1] = pltpu.async_copy(
                    table.at[srcb.at[j + 1]], rows_n, gsem_n)
        dsc[CHUNK - 2].wait()
        dsc[CHUNK - 1].wait()
        return _
    lax.fori_loop(0, NCH, _chunk, None)

    plsc.subcore_barrier()
    pltpu.sync_copy(acc.at[pl.ds(s * ZROWS, ZROWS)],
                    out.at[c].at[pl.ds(s * ZROWS, ZROWS)])


@functools.lru_cache(maxsize=1)
def _sc_agg_fn():
    mesh = plsc.VectorSubcoreMesh(
        core_axis_name="c", subcore_axis_name="s", num_cores=2,
        num_subcores=16)
    return pl.kernel(
        _sc_agg_body,
        out_type=jax.ShapeDtypeStruct((2, NP, DW), jnp.float32),
        mesh=mesh,
        compiler_params=pltpu.CompilerParams(use_tc_tiling_on_sc=False),
        scratch_types=[
            pltpu.VMEM((CHUNK, 128), jnp.int32),
            pltpu.VMEM((CHUNK, 128), jnp.int32),
            pltpu.VMEM((128, DW), jnp.float32),
            pltpu.VMEM((128, DW), jnp.float32),
            pltpu.VMEM_SHARED((NP, DW), jnp.float32),
            pltpu.SemaphoreType.DMA,
            pltpu.SemaphoreType.DMA,
            pltpu.SemaphoreType.DMA,
            pltpu.SemaphoreType.DMA,
        ],
    )


def _sc_agg(table, src2, dst2):
    return _sc_agg_fn()(table, src2, dst2)


def _enc_body(x_ref, w_ref, o_ref):
    h = jnp.dot(x_ref[...], w_ref[...], preferred_element_type=jnp.float32)
    col = lax.broadcasted_iota(jnp.int32, (BR, DW), 1)
    o_ref[...] = jnp.where(col == D, 1.0, h)


def _tc_encode(x_pad, W_encp):
    return pl.pallas_call(
        _enc_body,
        grid=(NP // BR,),
        in_specs=[
            pl.BlockSpec((BR, IN_CH), lambda i: (i, 0)),
            pl.BlockSpec((IN_CH, DW), lambda i: (0, 0)),
        ],
        out_specs=pl.BlockSpec((BR, DW), lambda i: (i, 0)),
        out_shape=jax.ShapeDtypeStruct((NP, DW), jnp.float32),
    )(x_pad, W_encp)


def _step_body(raw_ref, u_ref, w_ref, hh_ref, un_ref):
    r = raw_ref[0] + raw_ref[1]
    col = lax.broadcasted_iota(jnp.int32, (BR, DW), 1)
    deg = jnp.maximum(r[:, D:D + 1], 1.0)
    agg = jnp.where(col < D, r / deg, 0.0)
    v = agg
    s = (v >= VTH).astype(jnp.float32)
    ssum = s
    for _ in range(T - 1):
        v = v - s + agg
        s = (v >= VTH).astype(jnp.float32)
        ssum = ssum + s
    rate = ssum * (1.0 / T)
    t = u_ref[...] + STEP * rate
    t = jnp.where((col >= 1) & (col < D), t, 0.0)
    n = jnp.sqrt(jnp.maximum(jnp.sum(t * t, axis=1, keepdims=True), 1e-12))
    en = jnp.exp(n)
    ien = 1.0 / en
    ch = 0.5 * (en + ien)
    sh = 0.5 * (en - ien)
    zs = sh / n * t
    x0 = jnp.maximum(ch, 1.0 + 1e-7)
    nn = jnp.sqrt(jnp.maximum(jnp.sum(zs * zs, axis=1, keepdims=True), 1e-12))
    d = jnp.log(x0 + jnp.sqrt((x0 - 1.0) * (x0 + 1.0)))
    un = d * zs / nn
    hh = jnp.dot(un, w_ref[...], preferred_element_type=jnp.float32)
    hh_ref[...] = jnp.where(col == D, 1.0, hh)
    un_ref[...] = un


def _tc_step(raw, u_prev, Wp):
    return pl.pallas_call(
        _step_body,
        grid=(NP // BR,),
        in_specs=[
            pl.BlockSpec((2, BR, DW), lambda i: (0, i, 0)),
            pl.BlockSpec((BR, DW), lambda i: (i, 0)),
            pl.BlockSpec((DW, DW), lambda i: (0, 0)),
        ],
        out_specs=[
            pl.BlockSpec((BR, DW), lambda i: (i, 0)),
            pl.BlockSpec((BR, DW), lambda i: (i, 0)),
        ],
        out_shape=[
            jax.ShapeDtypeStruct((NP, DW), jnp.float32),
            jax.ShapeDtypeStruct((NP, DW), jnp.float32),
        ],
    )(raw, u_prev, Wp)


def kernel(x, edge_index, W_enc, W_l0, W_l1, W_fc):
    src = edge_index[0]
    dst = edge_index[1]
    npad = EP - E
    pad_src = jnp.arange(npad, dtype=jnp.int32) % N
    pad_dst = N + jnp.arange(npad, dtype=jnp.int32) % (NP - N)
    src2 = jnp.concatenate([src, pad_src]).reshape(IDX_ROWS, 128)
    dst2 = jnp.concatenate([dst, pad_dst]).reshape(IDX_ROWS, 128)

    x_pad = jnp.pad(x, ((0, NP - N), (0, 0)))
    W_encp = jnp.pad(W_enc, ((0, 0), (0, DW - D)))
    W0p = jnp.pad(W_l0, ((0, DW - D), (0, DW - D)))
    W1p = jnp.pad(W_l1, ((0, DW - D), (0, DW - D)))
    Wfcp = jnp.pad(W_fc, ((0, DW - D), (0, DW - OUT_CH)))

    h = _tc_encode(x_pad, W_encp)
    raw = _sc_agg(h, src2, dst2)
    u = jnp.zeros((NP, DW), jnp.float32)
    hh, u = _tc_step(raw, u, W0p)
    raw = _sc_agg(hh, src2, dst2)
    hh, u = _tc_step(raw, u, W1p)
    raw = _sc_agg(hh, src2, dst2)
    outp, _ = _tc_step(raw, u, Wfcp)
    return outp[:N, :OUT_CH]


# speedup vs baseline: 16.2428x; 1.3778x over previous
import functools

import jax
import jax.numpy as jnp
from jax import lax
from jax.experimental import pallas as pl
from jax.experimental import layout as jlayout
from jax.experimental.pallas import tpu as pltpu
from jax.experimental.pallas import tpu_sc as plsc

N = 10000
E = 320000
IN_CH = 128
D = 33
OUT_CH = 16
T = 4
STEP = 0.1
VTH = 1.0

DW = 48
NP = 10240
EP = 327680
IDX_ROWS = EP // 128
NW = 32
ROWS_PER_W = IDX_ROWS // NW
UROWS = 4
UNITS = ROWS_PER_W // UROWS
ZROWS = NP // 16
BR = 512

_LINEAR16 = jlayout.Layout(major_to_minor=(0, 1), tiling=((16,),))


def _sc_agg_body(table, src3, dst3, out, srcb, dstb, rowsa, rowsb, acc,
                 gsema, gsemb, ssema, ssemb):
    c = lax.axis_index("c")
    s = lax.axis_index("s")

    def _z(i, _):
        zero16 = jnp.zeros((16,), jnp.float32)
        for k in range(DW // 16):
            rowsa[i, pl.ds(k * 16, 16)] = zero16
        return _
    lax.fori_loop(0, 128, _z, None)
    for k in range(ZROWS // 128):
        pltpu.sync_copy(rowsa.at[pl.ds(0, 128)],
                        acc.at[pl.ds(s * ZROWS + k * 128, 128)])
    plsc.subcore_barrier()

    ubase = (c * 16 + s) * UNITS
    pltpu.sync_copy(src3.at[pl.ds(ubase, UNITS)], srcb)
    pltpu.sync_copy(dst3.at[pl.ds(ubase, UNITS)], dstb)
    dg = {}
    dsc = {}
    dg[0] = pltpu.async_copy(table.at[srcb.at[0]], rowsa, gsema)
    for k in range(UNITS):
        even = (k % 2 == 0)
        rows_k = rowsa if even else rowsb
        ssem_k = ssema if even else ssemb
        dg[k].wait()
        dsc[k] = pltpu.async_copy(rows_k, acc.at[dstb.at[k]], ssem_k,
                                  add=True)
        if k < UNITS - 1:
            rows_n = rowsb if even else rowsa
            gsem_n = gsemb if even else gsema
            if k >= 1:
                dsc[k - 1].wait()
            dg[k + 1] = pltpu.async_copy(
                table.at[srcb.at[k + 1]], rows_n, gsem_n)
    dsc[UNITS - 2].wait()
    dsc[UNITS - 1].wait()

    plsc.subcore_barrier()
    pltpu.sync_copy(acc.at[pl.ds(s * ZROWS, ZROWS)],
                    out.at[c].at[pl.ds(s * ZROWS, ZROWS)])


@functools.lru_cache(maxsize=1)
def _sc_agg_fn():
    mesh = plsc.VectorSubcoreMesh(
        core_axis_name="c", subcore_axis_name="s", num_cores=2,
        num_subcores=16)
    return pl.kernel(
        _sc_agg_body,
        out_type=jax.ShapeDtypeStruct((2, NP, DW), jnp.float32),
        mesh=mesh,
        compiler_params=pltpu.CompilerParams(use_tc_tiling_on_sc=False),
        scratch_types=[
            pltpu.VMEM((UNITS, UROWS * 128), jnp.int32),
            pltpu.VMEM((UNITS, UROWS * 128), jnp.int32),
            pltpu.VMEM((UROWS * 128, DW), jnp.float32),
            pltpu.VMEM((UROWS * 128, DW), jnp.float32),
            pltpu.VMEM_SHARED((NP, DW), jnp.float32),
            pltpu.SemaphoreType.DMA,
            pltpu.SemaphoreType.DMA,
            pltpu.SemaphoreType.DMA,
            pltpu.SemaphoreType.DMA,
        ],
    )


def _sc_agg(table, src2, dst2):
    return _sc_agg_fn()(table, src2, dst2)


def _enc_body(x_ref, w_ref, o_ref):
    h = jnp.dot(x_ref[...], w_ref[...], preferred_element_type=jnp.float32)
    col = lax.broadcasted_iota(jnp.int32, (BR, DW), 1)
    o_ref[...] = jnp.where(col == D, 1.0, h)


def _tc_encode(x_pad, W_encp):
    return pl.pallas_call(
        _enc_body,
        grid=(NP // BR,),
        in_specs=[
            pl.BlockSpec((BR, IN_CH), lambda i: (i, 0)),
            pl.BlockSpec((IN_CH, DW), lambda i: (0, 0)),
        ],
        out_specs=pl.BlockSpec((BR, DW), lambda i: (i, 0)),
        out_shape=jax.ShapeDtypeStruct((NP, DW), jnp.float32),
    )(x_pad, W_encp)


def _step_body(raw_ref, u_ref, w_ref, hh_ref, un_ref):
    r = raw_ref[0] + raw_ref[1]
    col = lax.broadcasted_iota(jnp.int32, (BR, DW), 1)
    deg = jnp.maximum(r[:, D:D + 1], 1.0)
    agg = jnp.where(col < D, r / deg, 0.0)
    v = agg
    s = (v >= VTH).astype(jnp.float32)
    ssum = s
    for _ in range(T - 1):
        v = v - s + agg
        s = (v >= VTH).astype(jnp.float32)
        ssum = ssum + s
    rate = ssum * (1.0 / T)
    t = u_ref[...] + STEP * rate
    t = jnp.where((col >= 1) & (col < D), t, 0.0)
    n = jnp.sqrt(jnp.maximum(jnp.sum(t * t, axis=1, keepdims=True), 1e-12))
    en = jnp.exp(n)
    ien = 1.0 / en
    ch = 0.5 * (en + ien)
    sh = 0.5 * (en - ien)
    zs = sh / n * t
    x0 = jnp.maximum(ch, 1.0 + 1e-7)
    nn = jnp.sqrt(jnp.maximum(jnp.sum(zs * zs, axis=1, keepdims=True), 1e-12))
    d = jnp.log(x0 + jnp.sqrt((x0 - 1.0) * (x0 + 1.0)))
    un = d * zs / nn
    hh = jnp.dot(un, w_ref[...], preferred_element_type=jnp.float32)
    hh_ref[...] = jnp.where(col == D, 1.0, hh)
    un_ref[...] = un


def _tc_step(raw, u_prev, Wp):
    return pl.pallas_call(
        _step_body,
        grid=(NP // BR,),
        in_specs=[
            pl.BlockSpec((2, BR, DW), lambda i: (0, i, 0)),
            pl.BlockSpec((BR, DW), lambda i: (i, 0)),
            pl.BlockSpec((DW, DW), lambda i: (0, 0)),
        ],
        out_specs=[
            pl.BlockSpec((BR, DW), lambda i: (i, 0)),
            pl.BlockSpec((BR, DW), lambda i: (i, 0)),
        ],
        out_shape=[
            jax.ShapeDtypeStruct((NP, DW), jnp.float32),
            jax.ShapeDtypeStruct((NP, DW), jnp.float32),
        ],
    )(raw, u_prev, Wp)


def kernel(x, edge_index, W_enc, W_l0, W_l1, W_fc):
    src = edge_index[0]
    dst = edge_index[1]
    npad = EP - E
    pad_src = jnp.arange(npad, dtype=jnp.int32) % N
    pad_dst = N + jnp.arange(npad, dtype=jnp.int32) % (NP - N)
    src3 = jnp.concatenate([src, pad_src]).reshape(IDX_ROWS // UROWS,
                                                   UROWS * 128)
    dst3 = jnp.concatenate([dst, pad_dst]).reshape(IDX_ROWS // UROWS,
                                                   UROWS * 128)

    x_pad = jnp.pad(x, ((0, NP - N), (0, 0)))
    W_encp = jnp.pad(W_enc, ((0, 0), (0, DW - D)))
    W0p = jnp.pad(W_l0, ((0, DW - D), (0, DW - D)))
    W1p = jnp.pad(W_l1, ((0, DW - D), (0, DW - D)))
    Wfcp = jnp.pad(W_fc, ((0, DW - D), (0, DW - OUT_CH)))

    h = _tc_encode(x_pad, W_encp)
    raw = _sc_agg(h, src3, dst3)
    u = jnp.zeros((NP, DW), jnp.float32)
    hh, u = _tc_step(raw, u, W0p)
    raw = _sc_agg(hh, src3, dst3)
    hh, u = _tc_step(raw, u, W1p)
    raw = _sc_agg(hh, src3, dst3)
    outp, _ = _tc_step(raw, u, Wfcp)
    return outp[:N, :OUT_CH]


# speedup vs baseline: 16.5435x; 1.0185x over previous
import functools

import jax
import jax.numpy as jnp
from jax import lax
from jax.experimental import pallas as pl
from jax.experimental import layout as jlayout
from jax.experimental.pallas import tpu as pltpu
from jax.experimental.pallas import tpu_sc as plsc

N = 10000
E = 320000
IN_CH = 128
D = 33
OUT_CH = 16
T = 4
STEP = 0.1
VTH = 1.0

DW = 48
NP = 10240
EP = 327680
IDX_ROWS = EP // 128
NW = 32
ROWS_PER_W = IDX_ROWS // NW
UROWS = 5
UNITS = ROWS_PER_W // UROWS
ZROWS = NP // 16
BR = 512

_LINEAR16 = jlayout.Layout(major_to_minor=(0, 1), tiling=((16,),))


def _sc_agg_body(table, src3, dst3, out, srcb, dstb, rowsa, rowsb, acc,
                 gsema, gsemb, ssema, ssemb):
    c = lax.axis_index("c")
    s = lax.axis_index("s")

    def _z(i, _):
        zero16 = jnp.zeros((16,), jnp.float32)
        for k in range(DW // 16):
            rowsa[i, pl.ds(k * 16, 16)] = zero16
        return _
    lax.fori_loop(0, 128, _z, None)
    for k in range(ZROWS // 128):
        pltpu.sync_copy(rowsa.at[pl.ds(0, 128)],
                        acc.at[pl.ds(s * ZROWS + k * 128, 128)])
    plsc.subcore_barrier()

    ubase = (c * 16 + s) * UNITS
    pltpu.sync_copy(src3.at[pl.ds(ubase, UNITS)], srcb)
    pltpu.sync_copy(dst3.at[pl.ds(ubase, UNITS)], dstb)
    dg = {}
    dsc = {}
    dg[0] = pltpu.async_copy(table.at[srcb.at[0]], rowsa, gsema)
    for k in range(UNITS):
        even = (k % 2 == 0)
        rows_k = rowsa if even else rowsb
        ssem_k = ssema if even else ssemb
        dg[k].wait()
        dsc[k] = pltpu.async_copy(rows_k, acc.at[dstb.at[k]], ssem_k,
                                  add=True)
        if k < UNITS - 1:
            rows_n = rowsb if even else rowsa
            gsem_n = gsemb if even else gsema
            if k >= 1:
                dsc[k - 1].wait()
            dg[k + 1] = pltpu.async_copy(
                table.at[srcb.at[k + 1]], rows_n, gsem_n)
    dsc[UNITS - 2].wait()
    dsc[UNITS - 1].wait()

    plsc.subcore_barrier()
    pltpu.sync_copy(acc.at[pl.ds(s * ZROWS, ZROWS)],
                    out.at[c].at[pl.ds(s * ZROWS, ZROWS)])


@functools.lru_cache(maxsize=1)
def _sc_agg_fn():
    mesh = plsc.VectorSubcoreMesh(
        core_axis_name="c", subcore_axis_name="s", num_cores=2,
        num_subcores=16)
    return pl.kernel(
        _sc_agg_body,
        out_type=jax.ShapeDtypeStruct((2, NP, DW), jnp.float32),
        mesh=mesh,
        compiler_params=pltpu.CompilerParams(use_tc_tiling_on_sc=False),
        scratch_types=[
            pltpu.VMEM((UNITS, UROWS * 128), jnp.int32),
            pltpu.VMEM((UNITS, UROWS * 128), jnp.int32),
            pltpu.VMEM((UROWS * 128, DW), jnp.float32),
            pltpu.VMEM((UROWS * 128, DW), jnp.float32),
            pltpu.VMEM_SHARED((NP, DW), jnp.float32),
            pltpu.SemaphoreType.DMA,
            pltpu.SemaphoreType.DMA,
            pltpu.SemaphoreType.DMA,
            pltpu.SemaphoreType.DMA,
        ],
    )


def _sc_agg(table, src2, dst2):
    return _sc_agg_fn()(table, src2, dst2)


def _enc_body(x_ref, w_ref, o_ref):
    h = jnp.dot(x_ref[...], w_ref[...], preferred_element_type=jnp.float32)
    col = lax.broadcasted_iota(jnp.int32, (BR, DW), 1)
    o_ref[...] = jnp.where(col == D, 1.0, h)


def _tc_encode(x_pad, W_encp):
    return pl.pallas_call(
        _enc_body,
        grid=(NP // BR,),
        in_specs=[
            pl.BlockSpec((BR, IN_CH), lambda i: (i, 0)),
            pl.BlockSpec((IN_CH, DW), lambda i: (0, 0)),
        ],
        out_specs=pl.BlockSpec((BR, DW), lambda i: (i, 0)),
        out_shape=jax.ShapeDtypeStruct((NP, DW), jnp.float32),
    )(x_pad, W_encp)


def _step_body(raw_ref, u_ref, w_ref, hh_ref, un_ref):
    r = raw_ref[0] + raw_ref[1]
    col = lax.broadcasted_iota(jnp.int32, (BR, DW), 1)
    deg = jnp.maximum(r[:, D:D + 1], 1.0)
    agg = jnp.where(col < D, r / deg, 0.0)
    v = agg
    s = (v >= VTH).astype(jnp.float32)
    ssum = s
    for _ in range(T - 1):
        v = v - s + agg
        s = (v >= VTH).astype(jnp.float32)
        ssum = ssum + s
    rate = ssum * (1.0 / T)
    t = u_ref[...] + STEP * rate
    t = jnp.where((col >= 1) & (col < D), t, 0.0)
    n = jnp.sqrt(jnp.maximum(jnp.sum(t * t, axis=1, keepdims=True), 1e-12))
    en = jnp.exp(n)
    ien = 1.0 / en
    ch = 0.5 * (en + ien)
    sh = 0.5 * (en - ien)
    zs = sh / n * t
    x0 = jnp.maximum(ch, 1.0 + 1e-7)
    nn = jnp.sqrt(jnp.maximum(jnp.sum(zs * zs, axis=1, keepdims=True), 1e-12))
    d = jnp.log(x0 + jnp.sqrt((x0 - 1.0) * (x0 + 1.0)))
    un = d * zs / nn
    hh = jnp.dot(un, w_ref[...], preferred_element_type=jnp.float32)
    hh_ref[...] = jnp.where(col == D, 1.0, hh)
    un_ref[...] = un


def _tc_step(raw, u_prev, Wp):
    return pl.pallas_call(
        _step_body,
        grid=(NP // BR,),
        in_specs=[
            pl.BlockSpec((2, BR, DW), lambda i: (0, i, 0)),
            pl.BlockSpec((BR, DW), lambda i: (i, 0)),
            pl.BlockSpec((DW, DW), lambda i: (0, 0)),
        ],
        out_specs=[
            pl.BlockSpec((BR, DW), lambda i: (i, 0)),
            pl.BlockSpec((BR, DW), lambda i: (i, 0)),
        ],
        out_shape=[
            jax.ShapeDtypeStruct((NP, DW), jnp.float32),
            jax.ShapeDtypeStruct((NP, DW), jnp.float32),
        ],
    )(raw, u_prev, Wp)


def kernel(x, edge_index, W_enc, W_l0, W_l1, W_fc):
    src = edge_index[0]
    dst = edge_index[1]
    npad = EP - E
    pad_src = jnp.arange(npad, dtype=jnp.int32) % N
    pad_dst = N + jnp.arange(npad, dtype=jnp.int32) % (NP - N)
    src3 = jnp.concatenate([src, pad_src]).reshape(IDX_ROWS // UROWS,
                                                   UROWS * 128)
    dst3 = jnp.concatenate([dst, pad_dst]).reshape(IDX_ROWS // UROWS,
                                                   UROWS * 128)

    x_pad = jnp.pad(x, ((0, NP - N), (0, 0)))
    W_encp = jnp.pad(W_enc, ((0, 0), (0, DW - D)))
    W0p = jnp.pad(W_l0, ((0, DW - D), (0, DW - D)))
    W1p = jnp.pad(W_l1, ((0, DW - D), (0, DW - D)))
    Wfcp = jnp.pad(W_fc, ((0, DW - D), (0, DW - OUT_CH)))

    h = _tc_encode(x_pad, W_encp)
    raw = _sc_agg(h, src3, dst3)
    u = jnp.zeros((NP, DW), jnp.float32)
    hh, u = _tc_step(raw, u, W0p)
    raw = _sc_agg(hh, src3, dst3)
    hh, u = _tc_step(raw, u, W1p)
    raw = _sc_agg(hh, src3, dst3)
    outp, _ = _tc_step(raw, u, Wfcp)
    return outp[:N, :OUT_CH]


# speedup vs baseline: 19.0863x; 1.1537x over previous
import functools

import jax
import jax.numpy as jnp
from jax import lax
from jax.experimental import pallas as pl
from jax.experimental.pallas import tpu as pltpu
from jax.experimental.pallas import tpu_sc as plsc

N = 10000
E = 320000
IN_CH = 128
D = 33
OUT_CH = 16
T = 4
STEP = 0.1
VTH = 1.0

DW = 64
NP = 10240
NH = NP // 2
EP = 327680
IDX_ROWS = EP // 128
NW = 32
ROWS_PER_W = IDX_ROWS // NW
UROWS = 4
UNITS = ROWS_PER_W // UROWS
ZROWS = NP // 16
BR = 512


def _sc_agg_body(table, src3, dst3, out, srcb, dstb, rowsa, rowsb, acc,
                 gsema, gsemb, ssema, ssemb):
    c = lax.axis_index("c")
    s = lax.axis_index("s")

    def _z(i, _):
        zero16 = jnp.zeros((16,), jnp.float32)
        for k in range(DW // 16):
            rowsa[i, pl.ds(k * 16, 16)] = zero16
        return _
    lax.fori_loop(0, 128, _z, None)
    for k in range(ZROWS // 128):
        pltpu.sync_copy(rowsa.at[pl.ds(0, 128)],
                        acc.at[pl.ds(s * ZROWS + k * 128, 128)])
    plsc.subcore_barrier()

    ubase = (c * 16 + s) * UNITS
    pltpu.sync_copy(src3.at[pl.ds(ubase, UNITS)], srcb)
    pltpu.sync_copy(dst3.at[pl.ds(ubase, UNITS)], dstb)
    dg = {}
    dsc = {}
    dg[0] = pltpu.async_copy(table.at[srcb.at[0]], rowsa, gsema)
    for k in range(UNITS):
        even = (k % 2 == 0)
        rows_k = rowsa if even else rowsb
        ssem_k = ssema if even else ssemb
        dg[k].wait()
        dsc[k] = pltpu.async_copy(rows_k, acc.at[dstb.at[k]], ssem_k,
                                  add=True)
        if k < UNITS - 1:
            rows_n = rowsb if even else rowsa
            gsem_n = gsemb if even else gsema
            if k >= 1:
                dsc[k - 1].wait()
            dg[k + 1] = pltpu.async_copy(
                table.at[srcb.at[k + 1]], rows_n, gsem_n)
    dsc[UNITS - 2].wait()
    dsc[UNITS - 1].wait()

    plsc.subcore_barrier()
    pltpu.sync_copy(acc.at[pl.ds(s * ZROWS, ZROWS)],
                    out.at[c].at[pl.ds(s * ZROWS, ZROWS)])


@functools.lru_cache(maxsize=1)
def _sc_agg_fn():
    mesh = plsc.VectorSubcoreMesh(
        core_axis_name="c", subcore_axis_name="s", num_cores=2,
        num_subcores=16)
    return pl.kernel(
        _sc_agg_body,
        out_type=jax.ShapeDtypeStruct((2, NP, DW), jnp.float32),
        mesh=mesh,
        compiler_params=pltpu.CompilerParams(use_tc_tiling_on_sc=False),
        scratch_types=[
            pltpu.VMEM((UNITS, UROWS * 128), jnp.int32),
            pltpu.VMEM((UNITS, UROWS * 128), jnp.int32),
            pltpu.VMEM((UROWS * 128, DW), jnp.float32),
            pltpu.VMEM((UROWS * 128, DW), jnp.float32),
            pltpu.VMEM_SHARED((NP, DW), jnp.float32),
            pltpu.SemaphoreType.DMA,
            pltpu.SemaphoreType.DMA,
            pltpu.SemaphoreType.DMA,
            pltpu.SemaphoreType.DMA,
        ],
    )


def _sc_agg(table_pair, src3, dst3):
    return _sc_agg_fn()(table_pair.reshape(NP, DW), src3, dst3)


def _enc_body(x_ref, w_ref, o_ref):
    xp = x_ref[...].reshape(BR, 2, IN_CH)
    h0 = jnp.dot(xp[:, 0, :], w_ref[...], preferred_element_type=jnp.float32)
    h1 = jnp.dot(xp[:, 1, :], w_ref[...], preferred_element_type=jnp.float32)
    h = jnp.concatenate([h0, h1], axis=1)
    col = lax.broadcasted_iota(jnp.int32, (BR, 2 * DW), 1)
    o_ref[...] = jnp.where((col & (DW - 1)) == D, 1.0, h)


def _tc_encode(x_pad, W_encp):
    return pl.pallas_call(
        _enc_body,
        grid=(NH // BR,),
        in_specs=[
            pl.BlockSpec((2 * BR, IN_CH), lambda i: (i, 0)),
            pl.BlockSpec((IN_CH, DW), lambda i: (0, 0)),
        ],
        out_specs=pl.BlockSpec((BR, 2 * DW), lambda i: (i, 0)),
        out_shape=jax.ShapeDtypeStruct((NH, 2 * DW), jnp.float32),
    )(x_pad, W_encp)


def _step_body(raw_ref, u_ref, w_ref, hh_ref, un_ref):
    r = raw_ref[0] + raw_ref[1]
    col = lax.broadcasted_iota(jnp.int32, (BR, 2 * DW), 1)
    cm = col & (DW - 1)
    left = col < DW
    deg = jnp.where(left, r[:, D:D + 1], r[:, DW + D:DW + D + 1])
    deg = jnp.maximum(deg, 1.0)
    agg = jnp.where(cm < D, r / deg, 0.0)
    v = agg
    s = (v >= VTH).astype(jnp.float32)
    ssum = s
    for _ in range(T - 1):
        v = v - s + agg
        s = (v >= VTH).astype(jnp.float32)
        ssum = ssum + s
    rate = ssum * (1.0 / T)
    t = u_ref[...] + STEP * rate
    t = jnp.where((cm >= 1) & (cm < D), t, 0.0)
    tt = t * t
    lm = left.astype(jnp.float32)
    n2l = jnp.sum(tt * lm, axis=1, keepdims=True)
    n2r = jnp.sum(tt * (1.0 - lm), axis=1, keepdims=True)
    n = jnp.sqrt(jnp.maximum(jnp.where(left, n2l, n2r), 1e-12))
    en = jnp.exp(n)
    ien = 1.0 / en
    ch = 0.5 * (en + ien)
    sh = 0.5 * (en - ien)
    zs = sh / n * t
    x0 = jnp.maximum(ch, 1.0 + 1e-7)
    zz = zs * zs
    m2l = jnp.sum(zz * lm, axis=1, keepdims=True)
    m2r = jnp.sum(zz * (1.0 - lm), axis=1, keepdims=True)
    nn = jnp.sqrt(jnp.maximum(jnp.where(left, m2l, m2r), 1e-12))
    d = jnp.log(x0 + jnp.sqrt((x0 - 1.0) * (x0 + 1.0)))
    un = d * zs / nn
    hh = jnp.dot(un, w_ref[...], preferred_element_type=jnp.float32)
    hh_ref[...] = jnp.where(cm == D, 1.0, hh)
    un_ref[...] = un


def _tc_step(raw, u_prev, W2p):
    return pl.pallas_call(
        _step_body,
        grid=(NH // BR,),
        in_specs=[
            pl.BlockSpec((2, BR, 2 * DW), lambda i: (0, i, 0)),
            pl.BlockSpec((BR, 2 * DW), lambda i: (i, 0)),
            pl.BlockSpec((2 * DW, 2 * DW), lambda i: (0, 0)),
        ],
        out_specs=[
            pl.BlockSpec((BR, 2 * DW), lambda i: (i, 0)),
            pl.BlockSpec((BR, 2 * DW), lambda i: (i, 0)),
        ],
        out_shape=[
            jax.ShapeDtypeStruct((NH, 2 * DW), jnp.float32),
            jax.ShapeDtypeStruct((NH, 2 * DW), jnp.float32),
        ],
    )(raw, u_prev, W2p)


def _blockdiag2(Wp):
    z = jnp.zeros((DW, DW), jnp.float32)
    return jnp.concatenate([
        jnp.concatenate([Wp, z], axis=1),
        jnp.concatenate([z, Wp], axis=1),
    ], axis=0)


def kernel(x, edge_index, W_enc, W_l0, W_l1, W_fc):
    src = edge_index[0]
    dst = edge_index[1]
    npad = EP - E
    pad_src = jnp.arange(npad, dtype=jnp.int32) % N
    pad_dst = N + jnp.arange(npad, dtype=jnp.int32) % (NP - N)
    src3 = jnp.concatenate([src, pad_src]).reshape(IDX_ROWS // UROWS,
                                                   UROWS * 128)
    dst3 = jnp.concatenate([dst, pad_dst]).reshape(IDX_ROWS // UROWS,
                                                   UROWS * 128)

    x_pad = jnp.pad(x, ((0, NP - N), (0, 0)))
    W_encp = jnp.pad(W_enc, ((0, 0), (0, DW - D)))
    W0p = jnp.pad(W_l0, ((0, DW - D), (0, DW - D)))
    W1p = jnp.pad(W_l1, ((0, DW - D), (0, DW - D)))
    Wfcp = jnp.pad(W_fc, ((0, DW - D), (0, DW - OUT_CH)))

    h = _tc_encode(x_pad, W_encp)
    raw = _sc_agg(h, src3, dst3).reshape(2, NH, 2 * DW)
    u = jnp.zeros((NH, 2 * DW), jnp.float32)
    hh, u = _tc_step(raw, u, _blockdiag2(W0p))
    raw = _sc_agg(hh, src3, dst3).reshape(2, NH, 2 * DW)
    hh, u = _tc_step(raw, u, _blockdiag2(W1p))
    raw = _sc_agg(hh, src3, dst3).reshape(2, NH, 2 * DW)
    outp, _ = _tc_step(raw, u, _blockdiag2(Wfcp))
    return outp.reshape(NP, DW)[:N, :OUT_CH]
